# baseline (speedup 1.0000x reference)
import functools

import jax
import jax.numpy as jnp
from jax import lax
from jax.experimental import pallas as pl
from jax.experimental.pallas import tpu as pltpu
from jax.experimental.pallas import tpu_sc as plsc

ENT_TOT = 1000000
REL_TOT = 1000
DIM = 64
B = 16384

NC = 2
NS = 16
L = 16
NW = NC * NS
BPW = B // NW
CH = 128
NCHUNK = BPW // CH
GP = CH // L


def _nrsqrt(x):
    xi = plsc.bitcast(x, jnp.int32)
    yi = jnp.int32(0x5F3759DF) - (xi >> 1)
    y = plsc.bitcast(yi, jnp.float32)
    half = x * jnp.float32(-0.5)
    for _ in range(3):
        y = y * (jnp.float32(1.5) + half * y * y)
    return y


def _scores_kernel(head_hbm, rel_hbm, tail_hbm, neg_hbm, ent_hbm, relemb_hbm,
                   pos_out, neg_out,
                   ih_v, ir_v, it_v, in_v,
                   hv, rv, tv, nv,
                   ps1, ps2, ns1, ns2, sem):
    wid = lax.axis_index("s") * NC + lax.axis_index("c")
    base = wid * BPW

    pltpu.sync_copy(head_hbm.at[pl.ds(base, BPW)], ih_v)
    pltpu.sync_copy(rel_hbm.at[pl.ds(base, BPW)], ir_v)
    pltpu.sync_copy(tail_hbm.at[pl.ds(base, BPW)], it_v)
    pltpu.sync_copy(neg_hbm.at[pl.ds(base, BPW)], in_v)

    row_iota = lax.iota(jnp.int32, L)

    for c in range(NCHUNK):
        cp1 = pltpu.async_copy(ent_hbm.at[ih_v.at[pl.ds(c * CH, CH)]], hv, sem)
        cp2 = pltpu.async_copy(relemb_hbm.at[ir_v.at[pl.ds(c * CH, CH)]], rv, sem)
        cp3 = pltpu.async_copy(ent_hbm.at[it_v.at[pl.ds(c * CH, CH)]], tv, sem)
        cp4 = pltpu.async_copy(ent_hbm.at[in_v.at[pl.ds(c * CH, CH)]], nv, sem)
        cp1.wait()
        cp2.wait()
        cp3.wait()
        cp4.wait()

        def group_body(g, _):
            rows = row_iota + g * L
            zero = jnp.zeros((L,), jnp.float32)

            def d_body(d, carry):
                hh, tt, nn, sa, sb, sc_, sd = carry
                col = jnp.full((L,), 0, jnp.int32) + d
                h = plsc.load_gather(hv, [rows, col])
                r = plsc.load_gather(rv, [rows, col])
                t = plsc.load_gather(tv, [rows, col])
                n = plsc.load_gather(nv, [rows, col])
                rt = r * t
                hrt = h * rt
                nrt = n * rt
                hrn = h * r * n
                hh = hh + h * h
                tt = tt + t * t
                nn = nn + n * n
                sa = sa + hrt
                sb = sb + hrt * hrt
                sc_ = sc_ + nrt
                sd = sd + hrn * hrn
                return (hh, tt, nn, sa, sb, sc_, sd)

            hh, tt, nn, sa, sb, sc_, sd = lax.fori_loop(
                0, DIM, d_body, (zero,) * 7)

            big = jnp.float32(1e12)
            inv_h = jnp.minimum(_nrsqrt(hh), big)
            inv_t = jnp.minimum(_nrsqrt(tt), big)
            inv_n = jnp.minimum(_nrsqrt(nn), big)
            norm_b = sb * _nrsqrt(sb)
            norm_d = sd * _nrsqrt(sd)
            ht = inv_h * inv_t
            off = c * CH + g * L
            ps1[pl.ds(off, L)] = -(sa * ht)
            ps2[pl.ds(off, L)] = -(norm_b * ht)
            ns1[pl.ds(off, L)] = -(sc_ * inv_n * inv_t)
            ns2[pl.ds(off, L)] = -(norm_d * inv_h * inv_n)
            return 0

        lax.fori_loop(0, GP, group_body, 0)

    pltpu.sync_copy(ps1, pos_out.at[pl.ds(base, BPW)])
    pltpu.sync_copy(ps2, pos_out.at[pl.ds(B + base, BPW)])
    pltpu.sync_copy(ns1, neg_out.at[pl.ds(base, BPW)])
    pltpu.sync_copy(ns2, neg_out.at[pl.ds(B + base, BPW)])


_sc_call = functools.partial(
    pl.kernel,
    out_type=(
        jax.ShapeDtypeStruct((2 * B,), jnp.float32),
        jax.ShapeDtypeStruct((2 * B,), jnp.float32),
    ),
    mesh=plsc.VectorSubcoreMesh(core_axis_name="c", subcore_axis_name="s"),
    compiler_params=pltpu.CompilerParams(
        needs_layout_passes=False, use_tc_tiling_on_sc=False),
    scratch_types=[
        pltpu.VMEM((BPW,), jnp.int32),
        pltpu.VMEM((BPW,), jnp.int32),
        pltpu.VMEM((BPW,), jnp.int32),
        pltpu.VMEM((BPW,), jnp.int32),
        pltpu.VMEM((CH, DIM), jnp.float32),
        pltpu.VMEM((CH, DIM), jnp.float32),
        pltpu.VMEM((CH, DIM), jnp.float32),
        pltpu.VMEM((CH, DIM), jnp.float32),
        pltpu.VMEM((BPW,), jnp.float32),
        pltpu.VMEM((BPW,), jnp.float32),
        pltpu.VMEM((BPW,), jnp.float32),
        pltpu.VMEM((BPW,), jnp.float32),
        pltpu.SemaphoreType.DMA,
    ],
)(_scores_kernel)


def kernel(batch_head, batch_rel, batch_tail, batch_negative,
           ent_embeddings, rel_embeddings):
    return _sc_call(
        batch_head.astype(jnp.int32),
        batch_rel.astype(jnp.int32),
        batch_tail.astype(jnp.int32),
        batch_negative.astype(jnp.int32),
        ent_embeddings,
        rel_embeddings,
    )


# speedup vs baseline: 1.4920x; 1.4920x over previous
import functools

import jax
import jax.numpy as jnp
from jax import lax
from jax.experimental import pallas as pl
from jax.experimental.pallas import tpu as pltpu
from jax.experimental.pallas import tpu_sc as plsc

ENT_TOT = 1000000
REL_TOT = 1000
DIM = 64
B = 16384

NC = 2
NS = 16
L = 16
NW = NC * NS
BPW = B // NW
CH = 128
NCHUNK = BPW // CH
GP = CH // L


def _nrsqrt(x):
    xi = plsc.bitcast(x, jnp.int32)
    yi = jnp.int32(0x5F3759DF) - (xi >> 1)
    y = plsc.bitcast(yi, jnp.float32)
    half = x * jnp.float32(-0.5)
    for _ in range(3):
        y = y * (jnp.float32(1.5) + half * y * y)
    return y


def _scores_kernel(head_hbm, rel_hbm, tail_hbm, neg_hbm, ent_hbm, relemb_hbm,
                   pos_out, neg_out,
                   ihv, irv, itv, inv,
                   hv, rv, tv, nv,
                   ps1, ps2, ns1, ns2, sem):
    wid = lax.axis_index("s") * NC + lax.axis_index("c")
    base = wid * BPW

    row_iota = lax.iota(jnp.int32, L)

    for c in range(NCHUNK):
        cb = base + c * CH
        pltpu.sync_copy(head_hbm.at[pl.ds(cb, CH)], ihv)
        pltpu.sync_copy(rel_hbm.at[pl.ds(cb, CH)], irv)
        pltpu.sync_copy(tail_hbm.at[pl.ds(cb, CH)], itv)
        pltpu.sync_copy(neg_hbm.at[pl.ds(cb, CH)], inv)

        def fetch_body(g, _):
            vh = ihv[pl.ds(g * L, L)]
            vr = irv[pl.ds(g * L, L)]
            vt = itv[pl.ds(g * L, L)]
            vn = inv[pl.ds(g * L, L)]
            cps = []
            for j in range(L):
                dst = pl.ds(g * L + j, 1)
                cps.append(pltpu.async_copy(
                    ent_hbm.at[pl.ds(vh[j], 1)], hv.at[dst], sem))
                cps.append(pltpu.async_copy(
                    relemb_hbm.at[pl.ds(vr[j], 1)], rv.at[dst], sem))
                cps.append(pltpu.async_copy(
                    ent_hbm.at[pl.ds(vt[j], 1)], tv.at[dst], sem))
                cps.append(pltpu.async_copy(
                    ent_hbm.at[pl.ds(vn[j], 1)], nv.at[dst], sem))
            for cp in cps:
                cp.wait()
            return 0

        lax.fori_loop(0, GP, fetch_body, 0)

        def group_body(g, _):
            rows = row_iota + g * L
            zero = jnp.zeros((L,), jnp.float32)

            def d_body(d, carry):
                hh, tt, nn, sa, sb, sc_, sd = carry
                col = jnp.full((L,), 0, jnp.int32) + d
                h = plsc.load_gather(hv, [rows, col])
                r = plsc.load_gather(rv, [rows, col])
                t = plsc.load_gather(tv, [rows, col])
                n = plsc.load_gather(nv, [rows, col])
                rt = r * t
                hrt = h * rt
                nrt = n * rt
                hrn = h * r * n
                hh = hh + h * h
                tt = tt + t * t
                nn = nn + n * n
                sa = sa + hrt
                sb = sb + hrt * hrt
                sc_ = sc_ + nrt
                sd = sd + hrn * hrn
                return (hh, tt, nn, sa, sb, sc_, sd)

            hh, tt, nn, sa, sb, sc_, sd = lax.fori_loop(
                0, DIM, d_body, (zero,) * 7)

            big = jnp.float32(1e12)
            inv_h = jnp.minimum(_nrsqrt(hh), big)
            inv_t = jnp.minimum(_nrsqrt(tt), big)
            inv_n = jnp.minimum(_nrsqrt(nn), big)
            norm_b = sb * _nrsqrt(sb)
            norm_d = sd * _nrsqrt(sd)
            ht = inv_h * inv_t
            off = c * CH + g * L
            ps1[pl.ds(off, L)] = -(sa * ht)
            ps2[pl.ds(off, L)] = -(norm_b * ht)
            ns1[pl.ds(off, L)] = -(sc_ * inv_n * inv_t)
            ns2[pl.ds(off, L)] = -(norm_d * inv_h * inv_n)
            return 0

        lax.fori_loop(0, GP, group_body, 0)

    pltpu.sync_copy(ps1, pos_out.at[pl.ds(base, BPW)])
    pltpu.sync_copy(ps2, pos_out.at[pl.ds(B + base, BPW)])
    pltpu.sync_copy(ns1, neg_out.at[pl.ds(base, BPW)])
    pltpu.sync_copy(ns2, neg_out.at[pl.ds(B + base, BPW)])


_sc_call = functools.partial(
    pl.kernel,
    out_type=(
        jax.ShapeDtypeStruct((2 * B,), jnp.float32),
        jax.ShapeDtypeStruct((2 * B,), jnp.float32),
    ),
    mesh=plsc.VectorSubcoreMesh(core_axis_name="c", subcore_axis_name="s"),
    compiler_params=pltpu.CompilerParams(needs_layout_passes=False),
    scratch_types=[
        pltpu.VMEM((CH,), jnp.int32),
        pltpu.VMEM((CH,), jnp.int32),
        pltpu.VMEM((CH,), jnp.int32),
        pltpu.VMEM((CH,), jnp.int32),
        pltpu.VMEM((CH, DIM), jnp.float32),
        pltpu.VMEM((CH, DIM), jnp.float32),
        pltpu.VMEM((CH, DIM), jnp.float32),
        pltpu.VMEM((CH, DIM), jnp.float32),
        pltpu.VMEM((BPW,), jnp.float32),
        pltpu.VMEM((BPW,), jnp.float32),
        pltpu.VMEM((BPW,), jnp.float32),
        pltpu.VMEM((BPW,), jnp.float32),
        pltpu.SemaphoreType.DMA,
    ],
)(_scores_kernel)


def kernel(batch_head, batch_rel, batch_tail, batch_negative,
           ent_embeddings, rel_embeddings):
    return _sc_call(
        batch_head.astype(jnp.int32),
        batch_rel.astype(jnp.int32),
        batch_tail.astype(jnp.int32),
        batch_negative.astype(jnp.int32),
        ent_embeddings,
        rel_embeddings,
    )


# speedup vs baseline: 1.7626x; 1.1814x over previous
import functools

import jax
import jax.numpy as jnp
from jax import lax
from jax.experimental import pallas as pl
from jax.experimental.pallas import tpu as pltpu
from jax.experimental.pallas import tpu_sc as plsc

ENT_TOT = 1000000
REL_TOT = 1000
DIM = 64
B = 16384

NC = 2
NS = 16
L = 16
NW = NC * NS
BPW = B // NW
CH = 128
NCHUNK = BPW // CH
GP = CH // L


def _nrsqrt(x):
    xi = plsc.bitcast(x, jnp.int32)
    yi = jnp.int32(0x5F3759DF) - (xi >> 1)
    y = plsc.bitcast(yi, jnp.float32)
    half = x * jnp.float32(-0.5)
    for _ in range(3):
        y = y * (jnp.float32(1.5) + half * y * y)
    return y


def _scores_kernel(head_hbm, rel_hbm, tail_hbm, neg_hbm, ent_hbm, relemb_hbm,
                   pos_out, neg_out,
                   ihv, irv, itv, inv,
                   hv, rv, tv, nv,
                   ps1, ps2, ns1, ns2, sem):
    wid = lax.axis_index("s") * NC + lax.axis_index("c")
    base = wid * BPW

    row_iota = lax.iota(jnp.int32, L)

    for c in range(NCHUNK):
        cb = base + c * CH
        pltpu.sync_copy(head_hbm.at[pl.ds(cb, CH)], ihv)
        pltpu.sync_copy(rel_hbm.at[pl.ds(cb, CH)], irv)
        pltpu.sync_copy(tail_hbm.at[pl.ds(cb, CH)], itv)
        pltpu.sync_copy(neg_hbm.at[pl.ds(cb, CH)], inv)

        def fetch_body(g, _):
            vh = ihv[pl.ds(g * L, L)]
            vr = irv[pl.ds(g * L, L)]
            vt = itv[pl.ds(g * L, L)]
            vn = inv[pl.ds(g * L, L)]
            cps = []
            for j in range(L):
                dst = pl.ds(g * L + j, 1)
                cps.append(pltpu.async_copy(
                    ent_hbm.at[pl.ds(vh[j], 1)], hv.at[dst], sem))
                cps.append(pltpu.async_copy(
                    relemb_hbm.at[pl.ds(vr[j], 1)], rv.at[dst], sem))
                cps.append(pltpu.async_copy(
                    ent_hbm.at[pl.ds(vt[j], 1)], tv.at[dst], sem))
                cps.append(pltpu.async_copy(
                    ent_hbm.at[pl.ds(vn[j], 1)], nv.at[dst], sem))
            for cp in cps:
                cp.wait()
            return 0

        lax.fori_loop(0, GP, fetch_body, 0)

        def group_body(g, _):
            rows = row_iota + g * L
            zero = jnp.zeros((L,), jnp.float32)

            def d_body(d, carry):
                hh, tt, nn, sa, sb, sc_, sd = carry
                col = jnp.full((L,), 0, jnp.int32) + d
                h = plsc.load_gather(hv, [rows, col])
                r = plsc.load_gather(rv, [rows, col])
                t = plsc.load_gather(tv, [rows, col])
                n = plsc.load_gather(nv, [rows, col])
                rt = r * t
                hrt = h * rt
                nrt = n * rt
                hrn = h * r * n
                hh = hh + h * h
                tt = tt + t * t
                nn = nn + n * n
                sa = sa + hrt
                sb = sb + hrt * hrt
                sc_ = sc_ + nrt
                sd = sd + hrn * hrn
                return (hh, tt, nn, sa, sb, sc_, sd)

            hh, tt, nn, sa, sb, sc_, sd = lax.fori_loop(
                0, DIM, d_body, (zero,) * 7)

            big = jnp.float32(1e12)
            inv_h = jnp.minimum(_nrsqrt(hh), big)
            inv_t = jnp.minimum(_nrsqrt(tt), big)
            inv_n = jnp.minimum(_nrsqrt(nn), big)
            norm_b = sb * _nrsqrt(sb)
            norm_d = sd * _nrsqrt(sd)
            ht = inv_h * inv_t
            off = c * CH + g * L
            ps1[pl.ds(off, L)] = -(sa * ht)
            ps2[pl.ds(off, L)] = -(norm_b * ht)
            ns1[pl.ds(off, L)] = -(sc_ * inv_n * inv_t)
            ns2[pl.ds(off, L)] = -(norm_d * inv_h * inv_n)
            return 0

        lax.fori_loop(0, GP, group_body, 0)

    pltpu.sync_copy(ps1, pos_out.at[pl.ds(base, BPW)])
    pltpu.sync_copy(ps2, pos_out.at[pl.ds(B + base, BPW)])
    pltpu.sync_copy(ns1, neg_out.at[pl.ds(base, BPW)])
    pltpu.sync_copy(ns2, neg_out.at[pl.ds(B + base, BPW)])


_sc_call = functools.partial(
    pl.kernel,
    out_type=(
        jax.ShapeDtypeStruct((2 * B,), jnp.float32),
        jax.ShapeDtypeStruct((2 * B,), jnp.float32),
    ),
    mesh=plsc.VectorSubcoreMesh(core_axis_name="c", subcore_axis_name="s"),
    compiler_params=pltpu.CompilerParams(needs_layout_passes=False),
    scratch_types=[
        pltpu.VMEM((CH,), jnp.int32),
        pltpu.VMEM((CH,), jnp.int32),
        pltpu.VMEM((CH,), jnp.int32),
        pltpu.VMEM((CH,), jnp.int32),
        pltpu.VMEM((CH, DIM), jnp.float32),
        pltpu.VMEM((CH, DIM), jnp.float32),
        pltpu.VMEM((CH, DIM), jnp.float32),
        pltpu.VMEM((CH, DIM), jnp.float32),
        pltpu.VMEM((BPW,), jnp.float32),
        pltpu.VMEM((BPW,), jnp.float32),
        pltpu.VMEM((BPW,), jnp.float32),
        pltpu.VMEM((BPW,), jnp.float32),
        pltpu.SemaphoreType.DMA,
    ],
)(_scores_kernel)


_TW = 8192


def _transpose_body(src_ref, dst_ref):
    dst_ref[...] = src_ref[...].T


_tc_transpose = pl.pallas_call(
    _transpose_body,
    grid=(pl.cdiv(ENT_TOT, _TW),),
    in_specs=[pl.BlockSpec((DIM, _TW), lambda g: (0, g))],
    out_specs=pl.BlockSpec((_TW, DIM), lambda g: (g, 0)),
    out_shape=jax.ShapeDtypeStruct((ENT_TOT, DIM), jnp.float32),
)


def kernel(batch_head, batch_rel, batch_tail, batch_negative,
           ent_embeddings, rel_embeddings):
    ent_rows = _tc_transpose(ent_embeddings.T)
    return _sc_call(
        batch_head.astype(jnp.int32),
        batch_rel.astype(jnp.int32),
        batch_tail.astype(jnp.int32),
        batch_negative.astype(jnp.int32),
        ent_rows,
        rel_embeddings,
    )
